# Initial kernel scaffold; baseline (speedup 1.0000x reference)
import jax
import jax.numpy as jnp
from jax.experimental import pallas as pl


def kernel(node_feat, nn_idx, nonempty_mask, emb, W0, b0, Wout, bout, Watt, batt):
    raise NotImplementedError("write your pallas kernel here")


# speedup vs baseline: 30.7721x; 30.7721x over previous
import functools

import jax
import jax.numpy as jnp
import numpy as np
from jax import lax
from jax.experimental import pallas as pl
from jax.experimental.pallas import tpu as pltpu
from jax.experimental.pallas import tpu_sc as plsc

EPS = float(np.finfo(np.float32).eps)
N = 10000
K = 16
E1 = 3
D = 128
J = K * E1
HW = E1 * D

NC, NS, L = 2, 16, 16
NW = NC * NS
NODES_PER_W = 320
NPAD = NW * NODES_PER_W
PH = 2
NODES_PER_PH = NODES_PER_W // PH
GROUPS_PER_PH = NODES_PER_PH // L


def _sc_hist_body(nf_hbm, idx_hbm, c_hbm, nf_v, idx_v, hist_v):
    wid = lax.axis_index("s") * NC + lax.axis_index("c")
    pltpu.sync_copy(nf_hbm, nf_v)
    iota = lax.iota(jnp.int32, L)
    ones = jnp.ones((L,), jnp.float32)
    zeros = jnp.zeros((L,), jnp.float32)

    def phase(ph, carry):
        base = wid * NODES_PER_W + ph * NODES_PER_PH
        pltpu.sync_copy(idx_hbm.at[pl.ds(base * J, NODES_PER_PH * J)], idx_v)

        def zrow(i, c):
            for cc in range(HW // L):
                hist_v[i, pl.ds(cc * L, L)] = zeros
            return c

        lax.fori_loop(0, NODES_PER_PH, zrow, 0)

        def group(g, c):
            rows = g * L + iota
            for j in range(J):
                nbr = idx_v[pl.ds(g * (L * J) + j * L, L)]
                cls = plsc.load_gather(nf_v, [nbr])
                cols = cls + (D * (j % E1))
                plsc.addupdate_scatter(hist_v, [rows, cols], ones)
            return c

        lax.fori_loop(0, GROUPS_PER_PH, group, 0)
        pltpu.sync_copy(hist_v, c_hbm.at[pl.ds(base, NODES_PER_PH)])
        return carry

    lax.fori_loop(0, PH, phase, 0)


@functools.cache
def _sc_hist():
    return pl.kernel(
        _sc_hist_body,
        out_type=jax.ShapeDtypeStruct((NPAD, HW), jnp.float32),
        mesh=plsc.VectorSubcoreMesh(core_axis_name="c", subcore_axis_name="s",
                                    num_cores=NC, num_subcores=NS),
        compiler_params=pltpu.CompilerParams(needs_layout_passes=False),
        scratch_types=[
            pltpu.VMEM((N,), jnp.int32),
            pltpu.VMEM((NODES_PER_PH * J,), jnp.int32),
            pltpu.VMEM((NODES_PER_PH, HW), jnp.float32),
        ],
    )


BLK = 1000


def _tc_body(c_ref, mask_ref, emb_ref, w0_ref, b0_ref, wout_ref, bout_ref,
             watt_ref, batt_ref, out_ref, acc_ref):
    b = pl.program_id(0)
    hp = jnp.zeros((BLK, D), jnp.float32)
    for e in range(E1):
        m_e = jnp.dot(emb_ref[:], w0_ref[e], preferred_element_type=jnp.float32)
        hp = hp + jnp.dot(c_ref[:, e * D:(e + 1) * D], m_e,
                          preferred_element_type=jnp.float32)
    hp = hp * (mask_ref[:] * (1.0 / K)) + b0_ref[:]
    h = jnp.maximum(hp, 0.0)
    nrm = jnp.sqrt(jnp.sum(h * h, axis=1, keepdims=True))
    h = h / (nrm + EPS)
    y = jnp.dot(h, wout_ref[:], preferred_element_type=jnp.float32) + bout_ref[:]
    att = jax.nn.sigmoid(
        jnp.sum(h * watt_ref[:], axis=1, keepdims=True) + batt_ref[0, 0])
    part = jnp.sum(att * y, axis=0, keepdims=True)

    @pl.when(b == 0)
    def _init():
        acc_ref[:] = jnp.zeros_like(acc_ref)

    acc_ref[:] = acc_ref[:] + part

    @pl.when(b == pl.num_programs(0) - 1)
    def _fin():
        out_ref[:] = acc_ref[:] * (1.0 / N)


_tc_head = pl.pallas_call(
    _tc_body,
    grid=(N // BLK,),
    in_specs=[
        pl.BlockSpec((BLK, HW), lambda b: (b, 0)),
        pl.BlockSpec((BLK, 1), lambda b: (b, 0)),
        pl.BlockSpec((D, D), lambda b: (0, 0)),
        pl.BlockSpec((E1, D, D), lambda b: (0, 0, 0)),
        pl.BlockSpec((1, D), lambda b: (0, 0)),
        pl.BlockSpec((D, D), lambda b: (0, 0)),
        pl.BlockSpec((1, D), lambda b: (0, 0)),
        pl.BlockSpec((1, D), lambda b: (0, 0)),
        pl.BlockSpec((1, 1), lambda b: (0, 0)),
    ],
    out_specs=pl.BlockSpec((1, D), lambda b: (0, 0)),
    out_shape=jax.ShapeDtypeStruct((1, D), jnp.float32),
    scratch_shapes=[pltpu.VMEM((1, D), jnp.float32)],
)


def kernel(node_feat, nn_idx, nonempty_mask, emb, W0, b0, Wout, bout, Watt, batt):
    nf = node_feat.reshape(N).astype(jnp.int32)
    idx = nn_idx.reshape(N, J)
    idx = jnp.pad(idx, ((0, NPAD - N), (0, 0)))
    idx = idx.reshape(NPAD // L, L, J).transpose(0, 2, 1).reshape(-1)

    counts = _sc_hist()(nf, idx)

    emb_p = jnp.zeros((D, D), jnp.float32).at[:emb.shape[0]].set(emb)
    w0r = W0.reshape(E1, D, D)
    mask2d = nonempty_mask.reshape(N, 1)
    score = _tc_head(
        counts[:N], mask2d, emb_p, w0r,
        b0.reshape(1, D), Wout, bout.reshape(1, D),
        Watt.reshape(1, D), batt.reshape(1, 1),
    )
    return score
